# baseline (speedup 1.0000x reference)
import jax
import jax.numpy as jnp
from jax.experimental import pallas as pl


B, N, D_IN, H, D_OUT = 64, 256, 128, 256, 128
BB = 8

_BATCHED = (((2,), (1,)), ((0,), (0,)))


def _fused_kernel(Gr_ref, x_ref, W1_ref, b1_ref, W2_ref, b2_ref,
                  W3_ref, b3_ref, Wout_ref, bout_ref, out_ref):
    Ab = Gr_ref[...].reshape(BB, N, N).astype(jnp.bfloat16)
    h = x_ref[...]

    def gin_layer(h, W_ref, b_ref):
        d = h.shape[-1]
        agg = jax.lax.dot_general(
            Ab, h.astype(jnp.bfloat16), _BATCHED,
            preferred_element_type=jnp.float32) + h
        hf = jnp.dot(agg.reshape(BB * N, d).astype(jnp.bfloat16),
                     W_ref[...].astype(jnp.bfloat16),
                     preferred_element_type=jnp.float32) + b_ref[...]
        return jax.nn.relu(hf).reshape(BB, N, H)

    h = gin_layer(h, W1_ref, b1_ref)
    h = gin_layer(h, W2_ref, b2_ref)
    h = gin_layer(h, W3_ref, b3_ref)

    hg = jnp.sum(h, axis=1)
    out_ref[...] = (
        jnp.dot(hg, Wout_ref[...], preferred_element_type=jnp.float32)
        + bout_ref[...])


def kernel(G, x, W1, b1, W2, b2, W3, b3, Wout, bout):
    Gr = G.reshape(B, 2 * N, 128)
    b1r = b1.reshape(1, H)
    b2r = b2.reshape(1, H)
    b3r = b3.reshape(1, H)
    boutr = bout.reshape(1, D_OUT)

    const = lambda shape: pl.BlockSpec(shape, lambda i: (0,) * len(shape))
    out = pl.pallas_call(
        _fused_kernel,
        grid=(B // BB,),
        in_specs=[
            pl.BlockSpec((BB, 2 * N, 128), lambda i: (i, 0, 0)),
            pl.BlockSpec((BB, N, D_IN), lambda i: (i, 0, 0)),
            const((D_IN, H)), const((1, H)),
            const((H, H)), const((1, H)),
            const((H, H)), const((1, H)),
            const((H, D_OUT)), const((1, D_OUT)),
        ],
        out_specs=pl.BlockSpec((BB, D_OUT), lambda i: (i, 0)),
        out_shape=jax.ShapeDtypeStruct((B, D_OUT), jnp.float32),
    )(Gr, x, W1, b1r, W2, b2r, W3, b3r, Wout, boutr)

    side_loss = jnp.asarray(0.0, dtype=jnp.float32)
    return (out, side_loss)


# speedup vs baseline: 1.0007x; 1.0007x over previous
import jax
import jax.numpy as jnp
from jax.experimental import pallas as pl


B, N, D_IN, H, D_OUT = 64, 256, 128, 256, 128
BB = 32

_BATCHED = (((2,), (1,)), ((0,), (0,)))


def _fused_kernel(Gr_ref, x_ref, W1_ref, b1_ref, W2_ref, b2_ref,
                  W3_ref, b3_ref, Wout_ref, bout_ref, out_ref):
    Ab = Gr_ref[...].reshape(BB, N, N).astype(jnp.bfloat16)
    h = x_ref[...]

    def gin_layer(h, W_ref, b_ref):
        d = h.shape[-1]
        agg = jax.lax.dot_general(
            Ab, h.astype(jnp.bfloat16), _BATCHED,
            preferred_element_type=jnp.float32) + h
        hf = jnp.dot(agg.reshape(BB * N, d).astype(jnp.bfloat16),
                     W_ref[...].astype(jnp.bfloat16),
                     preferred_element_type=jnp.float32) + b_ref[...]
        return jax.nn.relu(hf).reshape(BB, N, H)

    h = gin_layer(h, W1_ref, b1_ref)
    h = gin_layer(h, W2_ref, b2_ref)
    h = gin_layer(h, W3_ref, b3_ref)

    hg = jnp.sum(h, axis=1)
    out_ref[...] = (
        jnp.dot(hg, Wout_ref[...], preferred_element_type=jnp.float32)
        + bout_ref[...])


def kernel(G, x, W1, b1, W2, b2, W3, b3, Wout, bout):
    Gr = G.reshape(B, 2 * N, 128)
    b1r = b1.reshape(1, H)
    b2r = b2.reshape(1, H)
    b3r = b3.reshape(1, H)
    boutr = bout.reshape(1, D_OUT)

    const = lambda shape: pl.BlockSpec(shape, lambda i: (0,) * len(shape))
    out = pl.pallas_call(
        _fused_kernel,
        grid=(B // BB,),
        in_specs=[
            pl.BlockSpec((BB, 2 * N, 128), lambda i: (i, 0, 0)),
            pl.BlockSpec((BB, N, D_IN), lambda i: (i, 0, 0)),
            const((D_IN, H)), const((1, H)),
            const((H, H)), const((1, H)),
            const((H, H)), const((1, H)),
            const((H, D_OUT)), const((1, D_OUT)),
        ],
        out_specs=pl.BlockSpec((BB, D_OUT), lambda i: (i, 0)),
        out_shape=jax.ShapeDtypeStruct((B, D_OUT), jnp.float32),
    )(Gr, x, W1, b1r, W2, b2r, W3, b3r, Wout, boutr)

    side_loss = jnp.asarray(0.0, dtype=jnp.float32)
    return (out, side_loss)


# speedup vs baseline: 1.0370x; 1.0362x over previous
import jax
import jax.numpy as jnp
from jax.experimental import pallas as pl


B, N, D_IN, H, D_OUT = 64, 256, 128, 256, 128
BB = 16

_BATCHED = (((2,), (1,)), ((0,), (0,)))


def _fused_kernel(Gr_ref, x_ref, W1_ref, b1_ref, W2_ref, b2_ref,
                  W3_ref, b3_ref, Wout_ref, bout_ref, out_ref):
    Ab = Gr_ref[...].reshape(BB, N, N).astype(jnp.bfloat16)
    h = x_ref[...]

    def gin_layer(h, W_ref, b_ref):
        d = h.shape[-1]
        agg = jax.lax.dot_general(
            Ab, h.astype(jnp.bfloat16), _BATCHED,
            preferred_element_type=jnp.float32) + h
        hf = jnp.dot(agg.reshape(BB * N, d).astype(jnp.bfloat16),
                     W_ref[...].astype(jnp.bfloat16),
                     preferred_element_type=jnp.float32) + b_ref[...]
        return jax.nn.relu(hf).reshape(BB, N, H)

    h = gin_layer(h, W1_ref, b1_ref)
    h = gin_layer(h, W2_ref, b2_ref)
    h = gin_layer(h, W3_ref, b3_ref)

    hg = jnp.sum(h, axis=1)
    out_ref[...] = (
        jnp.dot(hg, Wout_ref[...], preferred_element_type=jnp.float32)
        + bout_ref[...])


def kernel(G, x, W1, b1, W2, b2, W3, b3, Wout, bout):
    Gr = G.reshape(B, 2 * N, 128)
    b1r = b1.reshape(1, H)
    b2r = b2.reshape(1, H)
    b3r = b3.reshape(1, H)
    boutr = bout.reshape(1, D_OUT)

    const = lambda shape: pl.BlockSpec(shape, lambda i: (0,) * len(shape))
    out = pl.pallas_call(
        _fused_kernel,
        grid=(B // BB,),
        in_specs=[
            pl.BlockSpec((BB, 2 * N, 128), lambda i: (i, 0, 0)),
            pl.BlockSpec((BB, N, D_IN), lambda i: (i, 0, 0)),
            const((D_IN, H)), const((1, H)),
            const((H, H)), const((1, H)),
            const((H, H)), const((1, H)),
            const((H, D_OUT)), const((1, D_OUT)),
        ],
        out_specs=pl.BlockSpec((BB, D_OUT), lambda i: (i, 0)),
        out_shape=jax.ShapeDtypeStruct((B, D_OUT), jnp.float32),
    )(Gr, x, W1, b1r, W2, b2r, W3, b3r, Wout, boutr)

    side_loss = jnp.asarray(0.0, dtype=jnp.float32)
    return (out, side_loss)


# speedup vs baseline: 1.0466x; 1.0093x over previous
import jax
import jax.numpy as jnp
from jax.experimental import pallas as pl
from jax.experimental.pallas import tpu as pltpu


B, N, D_IN, H, D_OUT = 64, 256, 128, 256, 128
BB = 16

_BATCHED = (((2,), (1,)), ((0,), (0,)))


def _fused_kernel(Gr_ref, x_ref, W1_ref, b1_ref, W2_ref, b2_ref,
                  W3_ref, b3_ref, Wout_ref, bout_ref, out_ref):
    Ab = Gr_ref[...].reshape(BB, N, N).astype(jnp.bfloat16)
    h = x_ref[...]

    def gin_layer(h, W_ref, b_ref):
        d = h.shape[-1]
        agg = jax.lax.dot_general(
            Ab, h.astype(jnp.bfloat16), _BATCHED,
            preferred_element_type=jnp.float32) + h
        hf = jnp.dot(agg.reshape(BB * N, d).astype(jnp.bfloat16),
                     W_ref[...].astype(jnp.bfloat16),
                     preferred_element_type=jnp.float32) + b_ref[...]
        return jax.nn.relu(hf).reshape(BB, N, H)

    h = gin_layer(h, W1_ref, b1_ref)
    h = gin_layer(h, W2_ref, b2_ref)
    h = gin_layer(h, W3_ref, b3_ref)

    hg = jnp.sum(h, axis=1)
    out_ref[...] = (
        jnp.dot(hg, Wout_ref[...], preferred_element_type=jnp.float32)
        + bout_ref[...])


def kernel(G, x, W1, b1, W2, b2, W3, b3, Wout, bout):
    Gr = G.reshape(B, 2 * N, 128)
    b1r = b1.reshape(1, H)
    b2r = b2.reshape(1, H)
    b3r = b3.reshape(1, H)
    boutr = bout.reshape(1, D_OUT)

    const = lambda shape: pl.BlockSpec(shape, lambda i: (0,) * len(shape))
    out = pl.pallas_call(
        _fused_kernel,
        grid=(B // BB,),
        in_specs=[
            pl.BlockSpec((BB, 2 * N, 128), lambda i: (i, 0, 0)),
            pl.BlockSpec((BB, N, D_IN), lambda i: (i, 0, 0)),
            const((D_IN, H)), const((1, H)),
            const((H, H)), const((1, H)),
            const((H, H)), const((1, H)),
            const((H, D_OUT)), const((1, D_OUT)),
        ],
        out_specs=pl.BlockSpec((BB, D_OUT), lambda i: (i, 0)),
        out_shape=jax.ShapeDtypeStruct((B, D_OUT), jnp.float32),
        compiler_params=pltpu.CompilerParams(
            dimension_semantics=("parallel",)),
    )(Gr, x, W1, b1r, W2, b2r, W3, b3r, Wout, boutr)

    side_loss = jnp.asarray(0.0, dtype=jnp.float32)
    return (out, side_loss)
